# baseline (speedup 1.0000x reference)
import jax
import jax.numpy as jnp
from jax.experimental import pallas as pl
from jax.experimental.pallas import tpu as pltpu


def _round_up(n: int, m: int) -> int:
    return ((n + m - 1) // m) * m


def _make_body(bp: int, S: int):
    M = bp * S

    def _body(x_ref, m_ref, w0_ref, b0_ref, w1_ref, b1_ref, o_ref):
        xb = x_ref[...].astype(jnp.bfloat16)
        h = jnp.tanh(
            jnp.dot(xb, w0_ref[...], preferred_element_type=jnp.float32)
            + b0_ref[...]
        )
        h = jnp.tanh(
            jnp.dot(h.astype(jnp.bfloat16), w1_ref[...],
                    preferred_element_type=jnp.float32)
            + b1_ref[...]
        ).astype(jnp.bfloat16)

        m = m_ref[...].astype(jnp.float32)
        col_b = jax.lax.broadcasted_iota(jnp.int32, (8, M), 1) // S
        row_b = jax.lax.broadcasted_iota(jnp.int32, (8, M), 0)
        msel = jnp.where(col_b == row_b, m.reshape(1, M), 0.0)
        pooled = jnp.dot(msel.astype(jnp.bfloat16), h,
                         preferred_element_type=jnp.float32)
        lens = jnp.maximum(jnp.sum(m, axis=2).reshape(bp, 1), 1.0)
        o_ref[...] = (pooled[:bp, :] / lens).reshape(o_ref.shape)

    return _body


def kernel(x, mask, w0, w1, b0, b1):
    B, S, D_in = x.shape
    H1 = w0.shape[1]
    H2 = w1.shape[1]

    Din_p, H1_p, H2_p = (_round_up(d, 128) for d in (D_in, H1, H2))

    w0p = jnp.zeros((Din_p, H1_p), jnp.bfloat16).at[:D_in, :H1].set(
        w0.astype(jnp.bfloat16))
    w1p = jnp.zeros((H1_p, H2_p), jnp.bfloat16).at[:H1, :H2].set(
        w1.astype(jnp.bfloat16))
    b0p = jnp.zeros((1, H1_p), jnp.float32).at[:, :H1].set(
        b0.reshape(1, -1).astype(jnp.float32))
    b1p = jnp.zeros((1, H2_p), jnp.float32).at[:, :H2].set(
        b1.reshape(1, -1).astype(jnp.float32))

    xp = x
    if Din_p != D_in or S % 8:
        Sp = _round_up(S, 8)
        xp = jnp.zeros((B, Sp, Din_p), x.dtype).at[:, :S, :D_in].set(x)
        mask = jnp.zeros((B, Sp), mask.dtype).at[:, :S].set(mask)
        S = Sp

    bp = 2 if (B % 2 == 0 and bp_rows_ok(S, Din_p)) else 1
    nsteps = B // bp

    x2 = xp.reshape(B * S, Din_p)
    m3 = mask.reshape(nsteps, bp, S).astype(jnp.float32)

    out = pl.pallas_call(
        _make_body(bp, S),
        out_shape=jax.ShapeDtypeStruct((nsteps, bp, H2_p), jnp.float32),
        grid_spec=pltpu.PrefetchScalarGridSpec(
            num_scalar_prefetch=0,
            grid=(nsteps,),
            in_specs=[
                pl.BlockSpec((bp * S, Din_p), lambda i: (i, 0)),
                pl.BlockSpec((1, bp, S), lambda i: (i, 0, 0)),
                pl.BlockSpec((Din_p, H1_p), lambda i: (0, 0)),
                pl.BlockSpec((1, H1_p), lambda i: (0, 0)),
                pl.BlockSpec((H1_p, H2_p), lambda i: (0, 0)),
                pl.BlockSpec((1, H2_p), lambda i: (0, 0)),
            ],
            out_specs=pl.BlockSpec((1, bp, H2_p), lambda i: (i, 0, 0)),
        ),
        compiler_params=pltpu.CompilerParams(
            dimension_semantics=("arbitrary",),
            vmem_limit_bytes=56 << 20,
        ),
    )(x2, m3, w0p, b0p, w1p, b1p)
    return out.reshape(B, H2_p)[:, :H2].astype(x.dtype)


def bp_rows_ok(S: int, Din_p: int) -> bool:
    return 2 * S * Din_p * 4 * 2 <= 32 << 20
